# speedup vs baseline: 2.1695x; 2.1695x over previous
import jax
import jax.numpy as jnp
from jax.experimental import pallas as pl


def kernel(x, table):
    B = x.shape[0]
    T, D = table.shape
    BLK = 256

    def body(t_ref, o_ref):
        o_ref[...] = jnp.broadcast_to(t_ref[...][None], (B, BLK, D))

    out = pl.pallas_call(
        body,
        grid=(T // BLK,),
        in_specs=[pl.BlockSpec((BLK, D), lambda i: (i, 0))],
        out_specs=pl.BlockSpec((B, BLK, D), lambda i: (0, i, 0)),
        out_shape=jax.ShapeDtypeStruct((B, T, D), jnp.float32),
    )(table)
    return out
